# baseline (speedup 1.0000x reference)
import functools

import jax
import jax.numpy as jnp
from jax import lax
from jax.experimental import pallas as pl
from jax.experimental.pallas import tpu as pltpu
from jax.experimental.pallas import tpu_sc as plsc

B, N = 256, 128
D_INV, D_EDGE = 128, 16
N_ATOM, N_BOND, N_CHARGE, N_EXTRA = 100, 5, 13, 16

NC, NS = 2, 16
NW = NC * NS
ROWS = B * N
ROWS_PER_W = ROWS // NW
J = 16
GROUPS = ROWS_PER_W // J


def _edge_body(adj_hbm, table_hbm, out_hbm, idx_v, rows_v, sem):
    wid = lax.axis_index("s") * NC + lax.axis_index("c")
    base_row = wid * ROWS_PER_W

    def group(g, carry):
        row0 = base_row + g * J
        pltpu.sync_copy(adj_hbm.at[pl.ds(row0, J)], idx_v)
        descs = [
            pltpu.async_copy(table_hbm.at[idx_v.at[j]], rows_v.at[j], sem)
            for j in range(J)
        ]
        for d in descs:
            d.wait()
        pltpu.sync_copy(rows_v, out_hbm.at[pl.ds(row0, J)])
        return carry

    lax.fori_loop(0, GROUPS, group, 0)


@functools.cache
def _edge_gather():
    return pl.kernel(
        _edge_body,
        out_type=jax.ShapeDtypeStruct((ROWS, N, D_EDGE), jnp.float32),
        mesh=plsc.VectorSubcoreMesh(core_axis_name="c", subcore_axis_name="s"),
        scratch_types=[
            pltpu.VMEM((J, N), jnp.int32),
            pltpu.VMEM((J, N, D_EDGE), jnp.float32),
            pltpu.SemaphoreType.DMA,
        ],
        compiler_params=pltpu.CompilerParams(use_tc_tiling_on_sc=False),
    )


def _inv_body(types_ref, charges_ref, extra_ref, ttab_ref, ctab_ref,
              w1_ref, w2_ref, b_ref, out_ref):
    tcol = types_ref[0]
    ccol = charges_ref[0]
    oh_t = (tcol == lax.broadcasted_iota(jnp.int32, (N, 128), 1)).astype(jnp.float32)
    oh_c = (ccol == lax.broadcasted_iota(jnp.int32, (N, 16), 1)).astype(jnp.float32)
    inv = (jnp.dot(oh_t, ttab_ref[...], preferred_element_type=jnp.float32)
           + jnp.dot(oh_c, ctab_ref[...], preferred_element_type=jnp.float32))
    extra = extra_ref[0]
    out = (jnp.dot(inv, w1_ref[...], preferred_element_type=jnp.float32)
           + jnp.dot(extra, w2_ref[...], preferred_element_type=jnp.float32)
           + b_ref[...])
    out_ref[0] = out


def _x_inv(types_t, charges_t, extra, ttab_pad, ctab_pad, w1, w2, b2d):
    return pl.pallas_call(
        _inv_body,
        grid=(B,),
        in_specs=[
            pl.BlockSpec((1, N, 1), lambda i: (i, 0, 0)),
            pl.BlockSpec((1, N, 1), lambda i: (i, 0, 0)),
            pl.BlockSpec((1, N, N_EXTRA), lambda i: (i, 0, 0)),
            pl.BlockSpec((128, D_INV), lambda i: (0, 0)),
            pl.BlockSpec((16, D_INV), lambda i: (0, 0)),
            pl.BlockSpec((D_INV, D_INV), lambda i: (0, 0)),
            pl.BlockSpec((N_EXTRA, D_INV), lambda i: (0, 0)),
            pl.BlockSpec((1, D_INV), lambda i: (0, 0)),
        ],
        out_specs=pl.BlockSpec((1, N, D_INV), lambda i: (i, 0, 0)),
        out_shape=jax.ShapeDtypeStruct((B, N, D_INV), jnp.float32),
    )(types_t, charges_t, extra, ttab_pad, ctab_pad, w1, w2, b2d)


def kernel(atom_types, atom_charges, adjacency, mask, extra_feats,
           atom_type_table, charge_table, bond_table, W, b):
    del mask
    x_edge = _edge_gather()(adjacency.reshape(ROWS, N), bond_table)
    x_edge = x_edge.reshape(B, N, N, D_EDGE)

    ttab_pad = jnp.zeros((128, D_INV), jnp.float32).at[:N_ATOM].set(atom_type_table)
    ctab_pad = jnp.zeros((16, D_INV), jnp.float32).at[:N_CHARGE].set(charge_table)
    x_inv = _x_inv(atom_types[:, :, None], atom_charges[:, :, None], extra_feats,
                   ttab_pad, ctab_pad, W[:D_INV], W[D_INV:], b.reshape(1, D_INV))
    return (x_inv, x_edge)


# speedup vs baseline: 1.0022x; 1.0022x over previous
import functools

import jax
import jax.numpy as jnp
from jax import lax
from jax.experimental import pallas as pl
from jax.experimental.pallas import tpu as pltpu
from jax.experimental.pallas import tpu_sc as plsc

B, N = 256, 128
D_INV, D_EDGE = 128, 16
N_ATOM, N_BOND, N_CHARGE, N_EXTRA = 100, 5, 13, 16

NC, NS = 2, 16
NW = NC * NS
ROWS = B * N
ROWS_PER_W = ROWS // NW
J = 16
GROUPS = ROWS_PER_W // J


E = B * N * N
E_PER_W = E // NW
CHUNK = J * N


def _edge_body(adj_hbm, table_hbm, out_hbm, idx_v, rows_v, sem):
    wid = lax.axis_index("s") * NC + lax.axis_index("c")
    base = wid * E_PER_W

    def group(g, carry):
        i0 = base + g * CHUNK
        pltpu.sync_copy(adj_hbm.at[pl.ds(i0, CHUNK)], idx_v)
        pltpu.async_copy(table_hbm.at[idx_v], rows_v, sem).wait()
        pltpu.sync_copy(rows_v, out_hbm.at[pl.ds(i0, CHUNK)])
        return carry

    lax.fori_loop(0, GROUPS, group, 0)


@functools.cache
def _edge_gather():
    return pl.kernel(
        _edge_body,
        out_type=jax.ShapeDtypeStruct((E, D_EDGE), jnp.float32),
        mesh=plsc.VectorSubcoreMesh(core_axis_name="c", subcore_axis_name="s"),
        scratch_types=[
            pltpu.VMEM((CHUNK,), jnp.int32),
            pltpu.VMEM((CHUNK, D_EDGE), jnp.float32),
            pltpu.SemaphoreType.DMA,
        ],
        compiler_params=pltpu.CompilerParams(use_tc_tiling_on_sc=False),
    )


def _inv_body(types_ref, charges_ref, extra_ref, ttab_ref, ctab_ref,
              w1_ref, w2_ref, b_ref, out_ref):
    tcol = types_ref[0]
    ccol = charges_ref[0]
    oh_t = (tcol == lax.broadcasted_iota(jnp.int32, (N, 128), 1)).astype(jnp.float32)
    oh_c = (ccol == lax.broadcasted_iota(jnp.int32, (N, 16), 1)).astype(jnp.float32)
    inv = (jnp.dot(oh_t, ttab_ref[...], preferred_element_type=jnp.float32)
           + jnp.dot(oh_c, ctab_ref[...], preferred_element_type=jnp.float32))
    extra = extra_ref[0]
    out = (jnp.dot(inv, w1_ref[...], preferred_element_type=jnp.float32)
           + jnp.dot(extra, w2_ref[...], preferred_element_type=jnp.float32)
           + b_ref[...])
    out_ref[0] = out


def _x_inv(types_t, charges_t, extra, ttab_pad, ctab_pad, w1, w2, b2d):
    return pl.pallas_call(
        _inv_body,
        grid=(B,),
        in_specs=[
            pl.BlockSpec((1, N, 1), lambda i: (i, 0, 0)),
            pl.BlockSpec((1, N, 1), lambda i: (i, 0, 0)),
            pl.BlockSpec((1, N, N_EXTRA), lambda i: (i, 0, 0)),
            pl.BlockSpec((128, D_INV), lambda i: (0, 0)),
            pl.BlockSpec((16, D_INV), lambda i: (0, 0)),
            pl.BlockSpec((D_INV, D_INV), lambda i: (0, 0)),
            pl.BlockSpec((N_EXTRA, D_INV), lambda i: (0, 0)),
            pl.BlockSpec((1, D_INV), lambda i: (0, 0)),
        ],
        out_specs=pl.BlockSpec((1, N, D_INV), lambda i: (i, 0, 0)),
        out_shape=jax.ShapeDtypeStruct((B, N, D_INV), jnp.float32),
    )(types_t, charges_t, extra, ttab_pad, ctab_pad, w1, w2, b2d)


def kernel(atom_types, atom_charges, adjacency, mask, extra_feats,
           atom_type_table, charge_table, bond_table, W, b):
    del mask
    x_edge = _edge_gather()(adjacency.reshape(E), bond_table)
    x_edge = x_edge.reshape(B, N, N, D_EDGE)

    ttab_pad = jnp.zeros((128, D_INV), jnp.float32).at[:N_ATOM].set(atom_type_table)
    ctab_pad = jnp.zeros((16, D_INV), jnp.float32).at[:N_CHARGE].set(charge_table)
    x_inv = _x_inv(atom_types[:, :, None], atom_charges[:, :, None], extra_feats,
                   ttab_pad, ctab_pad, W[:D_INV], W[D_INV:], b.reshape(1, D_INV))
    return (x_inv, x_edge)


# speedup vs baseline: 11.0220x; 10.9977x over previous
import functools

import jax
import jax.numpy as jnp
from jax import lax
from jax.experimental import pallas as pl
from jax.experimental.pallas import tpu as pltpu
from jax.experimental.pallas import tpu_sc as plsc

B, N = 256, 128
D_INV, D_EDGE = 128, 16
N_ATOM, N_BOND, N_CHARGE, N_EXTRA = 100, 5, 13, 16

NC, NS = 2, 16
NW = NC * NS
ROWS = B * N
ROWS_PER_W = ROWS // NW
J = 16
GROUPS = ROWS_PER_W // J


E = B * N * N
E_PER_W = E // NW
CHUNK = J * N


def _edge_body(adj_hbm, table_hbm, out_hbm, tab_sh, idx_v, rows_v, sem):
    sid = lax.axis_index("s")
    wid = sid * NC + lax.axis_index("c")
    base = wid * E_PER_W

    @pl.when(sid == 0)
    def _stage():
        pltpu.sync_copy(table_hbm, tab_sh)

    plsc.subcore_barrier()

    def group(g, carry):
        i0 = base + g * CHUNK
        pltpu.sync_copy(adj_hbm.at[pl.ds(i0, CHUNK)], idx_v)
        pltpu.async_copy(tab_sh.at[idx_v], rows_v, sem).wait()
        pltpu.sync_copy(rows_v, out_hbm.at[pl.ds(i0, CHUNK)])
        return carry

    lax.fori_loop(0, GROUPS, group, 0)


@functools.cache
def _edge_gather():
    return pl.kernel(
        _edge_body,
        out_type=jax.ShapeDtypeStruct((E, D_EDGE), jnp.float32),
        mesh=plsc.VectorSubcoreMesh(core_axis_name="c", subcore_axis_name="s"),
        scratch_types=[
            pltpu.VMEM_SHARED((N_BOND, D_EDGE), jnp.float32),
            pltpu.VMEM((CHUNK,), jnp.int32),
            pltpu.VMEM((CHUNK, D_EDGE), jnp.float32),
            pltpu.SemaphoreType.DMA,
        ],
        compiler_params=pltpu.CompilerParams(use_tc_tiling_on_sc=False),
    )


def _inv_body(types_ref, charges_ref, extra_ref, ttab_ref, ctab_ref,
              w1_ref, w2_ref, b_ref, out_ref):
    tcol = types_ref[0]
    ccol = charges_ref[0]
    oh_t = (tcol == lax.broadcasted_iota(jnp.int32, (N, 128), 1)).astype(jnp.float32)
    oh_c = (ccol == lax.broadcasted_iota(jnp.int32, (N, 16), 1)).astype(jnp.float32)
    inv = (jnp.dot(oh_t, ttab_ref[...], preferred_element_type=jnp.float32)
           + jnp.dot(oh_c, ctab_ref[...], preferred_element_type=jnp.float32))
    extra = extra_ref[0]
    out = (jnp.dot(inv, w1_ref[...], preferred_element_type=jnp.float32)
           + jnp.dot(extra, w2_ref[...], preferred_element_type=jnp.float32)
           + b_ref[...])
    out_ref[0] = out


def _x_inv(types_t, charges_t, extra, ttab_pad, ctab_pad, w1, w2, b2d):
    return pl.pallas_call(
        _inv_body,
        grid=(B,),
        in_specs=[
            pl.BlockSpec((1, N, 1), lambda i: (i, 0, 0)),
            pl.BlockSpec((1, N, 1), lambda i: (i, 0, 0)),
            pl.BlockSpec((1, N, N_EXTRA), lambda i: (i, 0, 0)),
            pl.BlockSpec((128, D_INV), lambda i: (0, 0)),
            pl.BlockSpec((16, D_INV), lambda i: (0, 0)),
            pl.BlockSpec((D_INV, D_INV), lambda i: (0, 0)),
            pl.BlockSpec((N_EXTRA, D_INV), lambda i: (0, 0)),
            pl.BlockSpec((1, D_INV), lambda i: (0, 0)),
        ],
        out_specs=pl.BlockSpec((1, N, D_INV), lambda i: (i, 0, 0)),
        out_shape=jax.ShapeDtypeStruct((B, N, D_INV), jnp.float32),
    )(types_t, charges_t, extra, ttab_pad, ctab_pad, w1, w2, b2d)


def kernel(atom_types, atom_charges, adjacency, mask, extra_feats,
           atom_type_table, charge_table, bond_table, W, b):
    del mask
    x_edge = _edge_gather()(adjacency.reshape(E), bond_table)
    x_edge = x_edge.reshape(B, N, N, D_EDGE)

    ttab_pad = jnp.zeros((128, D_INV), jnp.float32).at[:N_ATOM].set(atom_type_table)
    ctab_pad = jnp.zeros((16, D_INV), jnp.float32).at[:N_CHARGE].set(charge_table)
    x_inv = _x_inv(atom_types[:, :, None], atom_charges[:, :, None], extra_feats,
                   ttab_pad, ctab_pad, W[:D_INV], W[D_INV:], b.reshape(1, D_INV))
    return (x_inv, x_edge)


# speedup vs baseline: 11.1495x; 1.0116x over previous
import functools

import jax
import jax.numpy as jnp
from jax import lax
from jax.experimental import pallas as pl
from jax.experimental.pallas import tpu as pltpu
from jax.experimental.pallas import tpu_sc as plsc

B, N = 256, 128
D_INV, D_EDGE = 128, 16
N_ATOM, N_BOND, N_CHARGE, N_EXTRA = 100, 5, 13, 16

NC, NS = 2, 16
NW = NC * NS
ROWS = B * N
ROWS_PER_W = ROWS // NW
J = 16
GROUPS = ROWS_PER_W // J


E = B * N * N
E_PER_W = E // NW
CHUNK = J * N
NBUF = 2
PAIRS = GROUPS // NBUF


def _edge_body(adj_hbm, table_hbm, out_hbm, tab_sh, idx_v, rows_v,
               si0, si1, sg, ss0, ss1):
    sid = lax.axis_index("s")
    wid = sid * NC + lax.axis_index("c")
    base = wid * E_PER_W
    sem_i = (si0, si1)
    sem_s = (ss0, ss1)

    @pl.when(sid == 0)
    def _stage():
        pltpu.sync_copy(table_hbm, tab_sh)

    plsc.subcore_barrier()

    def adj_sl(g):
        return adj_hbm.at[pl.ds(base + g * CHUNK, CHUNK)]

    def out_sl(g):
        return out_hbm.at[pl.ds(base + g * CHUNK, CHUNK)]

    for s in range(NBUF):
        pltpu.async_copy(adj_sl(s), idx_v.at[s], sem_i[s])

    def pair(p, carry):
        for s in range(NBUF):
            g = p * NBUF + s
            pltpu.make_async_copy(adj_sl(g), idx_v.at[s], sem_i[s]).wait()

            @pl.when(p > 0)
            def _drain():
                pltpu.make_async_copy(rows_v.at[s], out_sl(g - NBUF),
                                      sem_s[s]).wait()

            pltpu.async_copy(tab_sh.at[idx_v.at[s]], rows_v.at[s], sg).wait()
            pltpu.async_copy(rows_v.at[s], out_sl(g), sem_s[s])

            @pl.when(p < PAIRS - 1)
            def _prefetch():
                pltpu.async_copy(adj_sl(g + NBUF), idx_v.at[s], sem_i[s])

        return carry

    lax.fori_loop(0, PAIRS, pair, 0)
    for s in range(NBUF):
        g_last = PAIRS * NBUF - NBUF + s
        pltpu.make_async_copy(rows_v.at[s], out_sl(g_last), sem_s[s]).wait()


@functools.cache
def _edge_gather():
    return pl.kernel(
        _edge_body,
        out_type=jax.ShapeDtypeStruct((E, D_EDGE), jnp.float32),
        mesh=plsc.VectorSubcoreMesh(core_axis_name="c", subcore_axis_name="s"),
        scratch_types=[
            pltpu.VMEM_SHARED((N_BOND, D_EDGE), jnp.float32),
            pltpu.VMEM((NBUF, CHUNK), jnp.int32),
            pltpu.VMEM((NBUF, CHUNK, D_EDGE), jnp.float32),
            pltpu.SemaphoreType.DMA,
            pltpu.SemaphoreType.DMA,
            pltpu.SemaphoreType.DMA,
            pltpu.SemaphoreType.DMA,
            pltpu.SemaphoreType.DMA,
        ],
        compiler_params=pltpu.CompilerParams(use_tc_tiling_on_sc=False),
    )


def _inv_body(types_ref, charges_ref, extra_ref, ttab_ref, ctab_ref,
              w1_ref, w2_ref, b_ref, out_ref):
    tcol = types_ref[0]
    ccol = charges_ref[0]
    oh_t = (tcol == lax.broadcasted_iota(jnp.int32, (N, 128), 1)).astype(jnp.float32)
    oh_c = (ccol == lax.broadcasted_iota(jnp.int32, (N, 16), 1)).astype(jnp.float32)
    inv = (jnp.dot(oh_t, ttab_ref[...], preferred_element_type=jnp.float32)
           + jnp.dot(oh_c, ctab_ref[...], preferred_element_type=jnp.float32))
    extra = extra_ref[0]
    out = (jnp.dot(inv, w1_ref[...], preferred_element_type=jnp.float32)
           + jnp.dot(extra, w2_ref[...], preferred_element_type=jnp.float32)
           + b_ref[...])
    out_ref[0] = out


def _x_inv(types_t, charges_t, extra, ttab_pad, ctab_pad, w1, w2, b2d):
    return pl.pallas_call(
        _inv_body,
        grid=(B,),
        in_specs=[
            pl.BlockSpec((1, N, 1), lambda i: (i, 0, 0)),
            pl.BlockSpec((1, N, 1), lambda i: (i, 0, 0)),
            pl.BlockSpec((1, N, N_EXTRA), lambda i: (i, 0, 0)),
            pl.BlockSpec((128, D_INV), lambda i: (0, 0)),
            pl.BlockSpec((16, D_INV), lambda i: (0, 0)),
            pl.BlockSpec((D_INV, D_INV), lambda i: (0, 0)),
            pl.BlockSpec((N_EXTRA, D_INV), lambda i: (0, 0)),
            pl.BlockSpec((1, D_INV), lambda i: (0, 0)),
        ],
        out_specs=pl.BlockSpec((1, N, D_INV), lambda i: (i, 0, 0)),
        out_shape=jax.ShapeDtypeStruct((B, N, D_INV), jnp.float32),
    )(types_t, charges_t, extra, ttab_pad, ctab_pad, w1, w2, b2d)


def kernel(atom_types, atom_charges, adjacency, mask, extra_feats,
           atom_type_table, charge_table, bond_table, W, b):
    del mask
    x_edge = _edge_gather()(adjacency.reshape(E), bond_table)
    x_edge = x_edge.reshape(B, N, N, D_EDGE)

    ttab_pad = jnp.zeros((128, D_INV), jnp.float32).at[:N_ATOM].set(atom_type_table)
    ctab_pad = jnp.zeros((16, D_INV), jnp.float32).at[:N_CHARGE].set(charge_table)
    x_inv = _x_inv(atom_types[:, :, None], atom_charges[:, :, None], extra_feats,
                   ttab_pad, ctab_pad, W[:D_INV], W[D_INV:], b.reshape(1, D_INV))
    return (x_inv, x_edge)


# speedup vs baseline: 11.3371x; 1.0168x over previous
import functools

import jax
import jax.numpy as jnp
from jax import lax
from jax.experimental import pallas as pl
from jax.experimental.pallas import tpu as pltpu
from jax.experimental.pallas import tpu_sc as plsc

B, N = 256, 128
D_INV, D_EDGE = 128, 16
N_ATOM, N_BOND, N_CHARGE, N_EXTRA = 100, 5, 13, 16

NC, NS = 2, 16
NW = NC * NS
ROWS = B * N
ROWS_PER_W = ROWS // NW
J = 16
GROUPS = ROWS_PER_W // J


MOL_PER_W = B // NW
GPM = N // J
NBUF = 2
PAIRS = GROUPS // NBUF


def _edge_body(adj_hbm, table_hbm, out_hbm, tab_sh, idx_v, rows_v,
               si0, si1, sg, ss0, ss1):
    sid = lax.axis_index("s")
    wid = sid * NC + lax.axis_index("c")
    sem_i = (si0, si1)
    sem_s = (ss0, ss1)

    @pl.when(sid == 0)
    def _stage():
        pltpu.sync_copy(table_hbm, tab_sh)

    plsc.subcore_barrier()

    def adj_sl(g):
        return adj_hbm.at[MOL_PER_W * wid + g // GPM, pl.ds((g % GPM) * J, J)]

    def out_sl(g):
        return out_hbm.at[MOL_PER_W * wid + g // GPM, pl.ds((g % GPM) * J, J)]

    for s in range(NBUF):
        pltpu.async_copy(adj_sl(s), idx_v.at[s], sem_i[s])

    def pair(p, carry):
        for s in range(NBUF):
            g = p * NBUF + s
            pltpu.make_async_copy(adj_sl(g), idx_v.at[s], sem_i[s]).wait()

            @pl.when(p > 0)
            def _drain():
                pltpu.make_async_copy(rows_v.at[s], out_sl(g - NBUF),
                                      sem_s[s]).wait()

            descs = [
                pltpu.async_copy(tab_sh.at[idx_v.at[s, j]], rows_v.at[s, j], sg)
                for j in range(J)
            ]
            for d in descs:
                d.wait()
            pltpu.async_copy(rows_v.at[s], out_sl(g), sem_s[s])

            @pl.when(p < PAIRS - 1)
            def _prefetch():
                pltpu.async_copy(adj_sl(g + NBUF), idx_v.at[s], sem_i[s])

        return carry

    lax.fori_loop(0, PAIRS, pair, 0)
    for s in range(NBUF):
        g_last = PAIRS * NBUF - NBUF + s
        pltpu.make_async_copy(rows_v.at[s], out_sl(g_last), sem_s[s]).wait()


@functools.cache
def _edge_gather():
    return pl.kernel(
        _edge_body,
        out_type=jax.ShapeDtypeStruct((B, N, N, D_EDGE), jnp.float32),
        mesh=plsc.VectorSubcoreMesh(core_axis_name="c", subcore_axis_name="s"),
        scratch_types=[
            pltpu.VMEM_SHARED((N_BOND, D_EDGE), jnp.float32),
            pltpu.VMEM((NBUF, J, N), jnp.int32),
            pltpu.VMEM((NBUF, J, N, D_EDGE), jnp.float32),
            pltpu.SemaphoreType.DMA,
            pltpu.SemaphoreType.DMA,
            pltpu.SemaphoreType.DMA,
            pltpu.SemaphoreType.DMA,
            pltpu.SemaphoreType.DMA,
        ],
        compiler_params=pltpu.CompilerParams(use_tc_tiling_on_sc=False),
    )


MB = 8
R = MB * N


def _inv_body(types_ref, charges_ref, extra_ref, ttab_ref, ctab_ref,
              w1_ref, w2_ref, b_ref, out_ref):
    tcol = types_ref[...].reshape(R, 1)
    ccol = charges_ref[...].reshape(R, 1)
    oh_t = (tcol == lax.broadcasted_iota(jnp.int32, (R, 128), 1)).astype(jnp.float32)
    oh_c = (ccol == lax.broadcasted_iota(jnp.int32, (R, 16), 1)).astype(jnp.float32)
    inv = (jnp.dot(oh_t, ttab_ref[...], preferred_element_type=jnp.float32)
           + jnp.dot(oh_c, ctab_ref[...], preferred_element_type=jnp.float32))
    extra = extra_ref[...].reshape(R, N_EXTRA)
    out = (jnp.dot(inv, w1_ref[...], preferred_element_type=jnp.float32)
           + jnp.dot(extra, w2_ref[...], preferred_element_type=jnp.float32)
           + b_ref[...])
    out_ref[...] = out.reshape(MB, N, D_INV)


def _x_inv(types3, charges3, extra, ttab_pad, ctab_pad, w1, w2, b2d):
    return pl.pallas_call(
        _inv_body,
        grid=(B // MB,),
        in_specs=[
            pl.BlockSpec((MB, N, 1), lambda i: (i, 0, 0)),
            pl.BlockSpec((MB, N, 1), lambda i: (i, 0, 0)),
            pl.BlockSpec((MB, N, N_EXTRA), lambda i: (i, 0, 0)),
            pl.BlockSpec((128, D_INV), lambda i: (0, 0)),
            pl.BlockSpec((16, D_INV), lambda i: (0, 0)),
            pl.BlockSpec((D_INV, D_INV), lambda i: (0, 0)),
            pl.BlockSpec((N_EXTRA, D_INV), lambda i: (0, 0)),
            pl.BlockSpec((1, D_INV), lambda i: (0, 0)),
        ],
        out_specs=pl.BlockSpec((MB, N, D_INV), lambda i: (i, 0, 0)),
        out_shape=jax.ShapeDtypeStruct((B, N, D_INV), jnp.float32),
    )(types3, charges3, extra, ttab_pad, ctab_pad, w1, w2, b2d)


def kernel(atom_types, atom_charges, adjacency, mask, extra_feats,
           atom_type_table, charge_table, bond_table, W, b):
    del mask
    x_edge = _edge_gather()(adjacency, bond_table)

    ttab_pad = jnp.zeros((128, D_INV), jnp.float32).at[:N_ATOM].set(atom_type_table)
    ctab_pad = jnp.zeros((16, D_INV), jnp.float32).at[:N_CHARGE].set(charge_table)
    x_inv = _x_inv(atom_types[:, :, None], atom_charges[:, :, None], extra_feats,
                   ttab_pad, ctab_pad, W[:D_INV], W[D_INV:], b.reshape(1, D_INV))
    return (x_inv, x_edge)


# speedup vs baseline: 130.4588x; 11.5072x over previous
import functools

import jax
import jax.numpy as jnp
from jax import lax
from jax.experimental import pallas as pl
from jax.experimental.pallas import tpu as pltpu
from jax.experimental.pallas import tpu_sc as plsc

B, N = 256, 128
D_INV, D_EDGE = 128, 16
N_ATOM, N_BOND, N_CHARGE, N_EXTRA = 100, 5, 13, 16

NC, NS = 2, 16
NW = NC * NS
ROWS = B * N
ROWS_PER_W = ROWS // NW
J = 16
GROUPS = ROWS_PER_W // J


MOL_PER_W = B // NW
GPM = N // J
NBUF = 2
PAIRS = GROUPS // NBUF


L = 16
NV = N // L
_GDN = lax.GatherDimensionNumbers(
    offset_dims=(), collapsed_slice_dims=(0,), start_index_map=(0,))


def _edge_body(adj_hbm, tabt_hbm, out_hbm, tab_v, idx_v, rows_v,
               si0, si1, ss0, ss1):
    sid = lax.axis_index("s")
    wid = sid * NC + lax.axis_index("c")
    sem_i = (si0, si1)
    sem_s = (ss0, ss1)

    pltpu.sync_copy(tabt_hbm, tab_v)
    tabs = [tab_v[k] for k in range(D_EDGE)]

    def adj_sl(g):
        return adj_hbm.at[MOL_PER_W * wid + g // GPM, pl.ds((g % GPM) * J, J)]

    def out_sl(g):
        return out_hbm.at[MOL_PER_W * wid + g // GPM, pl.ds((g % GPM) * J, J)]

    for s in range(NBUF):
        pltpu.async_copy(adj_sl(s), idx_v.at[s], sem_i[s])

    def pair(p, carry):
        for s in range(NBUF):
            g = p * NBUF + s
            pltpu.make_async_copy(adj_sl(g), idx_v.at[s], sem_i[s]).wait()

            @pl.when(p > 0)
            def _drain():
                pltpu.make_async_copy(rows_v.at[s], out_sl(g - NBUF),
                                      sem_s[s]).wait()

            def row_work(i, c):
                for v in range(NV):
                    a = idx_v[s, i, pl.ds(L * v, L)]
                    ai = a[:, None]
                    for k in range(D_EDGE):
                        vals = lax.gather(
                            tabs[k], ai, _GDN, slice_sizes=(1,),
                            mode=lax.GatherScatterMode.PROMISE_IN_BOUNDS)
                        rows_v[s, i, k, pl.ds(L * v, L)] = vals
                return c

            lax.fori_loop(0, J, row_work, 0)
            pltpu.async_copy(rows_v.at[s], out_sl(g), sem_s[s])

            @pl.when(p < PAIRS - 1)
            def _prefetch():
                pltpu.async_copy(adj_sl(g + NBUF), idx_v.at[s], sem_i[s])

        return carry

    lax.fori_loop(0, PAIRS, pair, 0)
    for s in range(NBUF):
        g_last = PAIRS * NBUF - NBUF + s
        pltpu.make_async_copy(rows_v.at[s], out_sl(g_last), sem_s[s]).wait()


@functools.cache
def _edge_gather():
    return pl.kernel(
        _edge_body,
        out_type=jax.ShapeDtypeStruct((B, N, D_EDGE, N), jnp.float32),
        mesh=plsc.VectorSubcoreMesh(core_axis_name="c", subcore_axis_name="s"),
        scratch_types=[
            pltpu.VMEM((D_EDGE, L), jnp.float32),
            pltpu.VMEM((NBUF, J, N), jnp.int32),
            pltpu.VMEM((NBUF, J, D_EDGE, N), jnp.float32),
            pltpu.SemaphoreType.DMA,
            pltpu.SemaphoreType.DMA,
            pltpu.SemaphoreType.DMA,
            pltpu.SemaphoreType.DMA,
        ],
        compiler_params=pltpu.CompilerParams(use_tc_tiling_on_sc=False),
    )


MB = 8
R = MB * N


def _inv_body(types_ref, charges_ref, extra_ref, ttab_ref, ctab_ref,
              w1_ref, w2_ref, b_ref, out_ref):
    tcol = types_ref[...].reshape(R, 1)
    ccol = charges_ref[...].reshape(R, 1)
    oh_t = (tcol == lax.broadcasted_iota(jnp.int32, (R, 128), 1)).astype(jnp.float32)
    oh_c = (ccol == lax.broadcasted_iota(jnp.int32, (R, 16), 1)).astype(jnp.float32)
    inv = (jnp.dot(oh_t, ttab_ref[...], preferred_element_type=jnp.float32)
           + jnp.dot(oh_c, ctab_ref[...], preferred_element_type=jnp.float32))
    extra = extra_ref[...].reshape(R, N_EXTRA)
    out = (jnp.dot(inv, w1_ref[...], preferred_element_type=jnp.float32)
           + jnp.dot(extra, w2_ref[...], preferred_element_type=jnp.float32)
           + b_ref[...])
    out_ref[...] = out.reshape(MB, N, D_INV)


def _x_inv(types3, charges3, extra, ttab_pad, ctab_pad, w1, w2, b2d):
    return pl.pallas_call(
        _inv_body,
        grid=(B // MB,),
        in_specs=[
            pl.BlockSpec((MB, N, 1), lambda i: (i, 0, 0)),
            pl.BlockSpec((MB, N, 1), lambda i: (i, 0, 0)),
            pl.BlockSpec((MB, N, N_EXTRA), lambda i: (i, 0, 0)),
            pl.BlockSpec((128, D_INV), lambda i: (0, 0)),
            pl.BlockSpec((16, D_INV), lambda i: (0, 0)),
            pl.BlockSpec((D_INV, D_INV), lambda i: (0, 0)),
            pl.BlockSpec((N_EXTRA, D_INV), lambda i: (0, 0)),
            pl.BlockSpec((1, D_INV), lambda i: (0, 0)),
        ],
        out_specs=pl.BlockSpec((MB, N, D_INV), lambda i: (i, 0, 0)),
        out_shape=jax.ShapeDtypeStruct((B, N, D_INV), jnp.float32),
    )(types3, charges3, extra, ttab_pad, ctab_pad, w1, w2, b2d)


def kernel(atom_types, atom_charges, adjacency, mask, extra_feats,
           atom_type_table, charge_table, bond_table, W, b):
    del mask
    tabt = jnp.zeros((D_EDGE, L), jnp.float32).at[:, :N_BOND].set(bond_table.T)
    x_edge_t = _edge_gather()(adjacency, tabt)
    x_edge = jnp.swapaxes(x_edge_t, 2, 3)

    ttab_pad = jnp.zeros((128, D_INV), jnp.float32).at[:N_ATOM].set(atom_type_table)
    ctab_pad = jnp.zeros((16, D_INV), jnp.float32).at[:N_CHARGE].set(charge_table)
    x_inv = _x_inv(atom_types[:, :, None], atom_charges[:, :, None], extra_feats,
                   ttab_pad, ctab_pad, W[:D_INV], W[D_INV:], b.reshape(1, D_INV))
    return (x_inv, x_edge)
